# baseline (speedup 1.0000x reference)
import functools

import jax
import jax.numpy as jnp
from jax import lax
from jax.experimental import pallas as pl
from jax.experimental.pallas import tpu as pltpu
from jax.experimental.pallas import tpu_sc as plsc

VOCAB = 100000
D = 64
B = 4096
H = 200

NC = 2
NS = 16
NW = NC * NS
BPW = B // NW
SPLIT_A = 104
SPLIT_B = H - SPLIT_A
NBUF = 4
UNROLL = 4


def _pool_sc(x_hbm, tbl_hbm, out_hbm, idx_v, rows_v, t_v, sems):
    wid = lax.axis_index("s") * NC + lax.axis_index("c")
    base = wid * BPW

    pltpu.sync_copy(x_hbm.at[pl.ds(base * H, BPW * H)], idx_v)

    def idx_view(i, lo, n):
        return idx_v.at[pl.ds(i * H + lo, n)]

    def start(i, b):
        pltpu.async_copy(tbl_hbm.at[idx_view(i, 0, SPLIT_A)],
                         rows_v.at[b, pl.ds(0, SPLIT_A), :], sems.at[b])
        pltpu.async_copy(tbl_hbm.at[idx_view(i, SPLIT_A, SPLIT_B)],
                         rows_v.at[b, pl.ds(SPLIT_A, SPLIT_B), :], sems.at[b])

    def wait(i, b):
        pltpu.make_async_copy(tbl_hbm.at[idx_view(i, 0, SPLIT_A)],
                              rows_v.at[b, pl.ds(0, SPLIT_A), :],
                              sems.at[b]).wait()
        pltpu.make_async_copy(tbl_hbm.at[idx_view(i, SPLIT_A, SPLIT_B)],
                              rows_v.at[b, pl.ds(SPLIT_A, SPLIT_B), :],
                              sems.at[b]).wait()

    for b in range(NBUF):
        start(b, b)

    zero = jnp.zeros((16,), jnp.float32)
    scale = jnp.float32(1.0 / H)

    def outer(i0, carry):
        for b in range(NBUF):
            i = i0 * NBUF + b
            wait(i, b)

            def rbody(r, acc):
                acc = list(acc)
                for u in range(UNROLL):
                    row = r * UNROLL + u
                    for c2 in range(2):
                        packed = rows_v[b, row, pl.ds(c2 * 32, 32)]
                        ea, eb = plsc.unpack(
                            packed, format=plsc.PackFormat.INTERLEAVED)
                        acc[c2 * 2] = acc[c2 * 2] + ea
                        acc[c2 * 2 + 1] = acc[c2 * 2 + 1] + eb
                return tuple(acc)

            acc = lax.fori_loop(0, H // UNROLL, rbody, (zero,) * 4)

            @pl.when(i + NBUF < BPW)
            def _():
                start(i + NBUF, b)

            for c2 in range(2):
                t_v[i, pl.ds(c2 * 32, 16)] = acc[c2 * 2] * scale
                t_v[i, pl.ds(c2 * 32 + 16, 16)] = acc[c2 * 2 + 1] * scale
        return carry

    lax.fori_loop(0, BPW // NBUF, outer, 0)

    pltpu.sync_copy(t_v, out_hbm.at[pl.ds(base, BPW), :])


@functools.partial(jax.jit, static_argnames=())
def _pool(x_flat, emb_table_bf16):
    mesh = plsc.VectorSubcoreMesh(core_axis_name="c", subcore_axis_name="s")
    f = pl.kernel(
        _pool_sc,
        mesh=mesh,
        out_type=jax.ShapeDtypeStruct((B, D), jnp.float32),
        scratch_types=[
            pltpu.VMEM((BPW * H,), jnp.int32),
            pltpu.VMEM((NBUF, H, D), jnp.bfloat16),
            pltpu.VMEM((BPW, D), jnp.float32),
            pltpu.SemaphoreType.DMA((NBUF,)),
        ],
        compiler_params=pltpu.CompilerParams(use_tc_tiling_on_sc=False,
                                             needs_layout_passes=False),
    )
    return f(x_flat, emb_table_bf16)


def _head_tc(t_ref, bio_ref, wb_ref, bb_ref, wh_ref, bh_ref,
             logits_ref, am_ref):
    stored = t_ref[...]
    s = lax.broadcasted_iota(jnp.int32, (D, D), 0)
    o = lax.broadcasted_iota(jnp.int32, (D, D), 1)
    orig = 32 * (s // 32) + 2 * (s % 16) + (s % 32) // 16
    perm = (orig == o).astype(jnp.float32)
    t = jnp.dot(stored, perm, preferred_element_type=jnp.float32)
    b = jnp.dot(bio_ref[...], wb_ref[...],
                preferred_element_type=jnp.float32) + bb_ref[...]
    attn = jax.nn.sigmoid(jnp.sum(t * b, axis=-1, keepdims=True))
    fused = t * attn + b * (1.0 - attn)
    logits_ref[...] = jnp.dot(fused, wh_ref[...],
                              preferred_element_type=jnp.float32) + bh_ref[...]
    am_ref[...] = jnp.mean(attn).reshape(1, 1)


def kernel(x, bio_features, emb_table, W_bio, b_bio, W_head, b_head):
    t = _pool(x.reshape(-1), emb_table.astype(jnp.bfloat16))
    logits, am = pl.pallas_call(
        _head_tc,
        out_shape=(
            jax.ShapeDtypeStruct((B, 2), jnp.float32),
            jax.ShapeDtypeStruct((1, 1), jnp.float32),
        ),
    )(t, bio_features, W_bio, b_bio.reshape(1, D), W_head,
      b_head.reshape(1, 2))
    return (logits, am[0, 0])
